# baseline (speedup 1.0000x reference)
import functools

import jax
import jax.numpy as jnp
from jax import lax
from jax.experimental import pallas as pl
from jax.experimental.pallas import tpu as pltpu
from jax.experimental.pallas import tpu_sc as plsc

_B, _S, _F, _K = 4, 2048, 1024, 64
_BR = 512

_NW = 32
_ROWS_PER_TILE = _B * _F // _NW
_CHUNK = 8


def _mean_topk_body(w_ref, idx_ref, acc_ref):
    b = pl.program_id(0)
    r = pl.program_id(1)
    nr = pl.num_programs(1)
    part = jnp.sum(w_ref[0], axis=0, keepdims=True)

    @pl.when(r == 0)
    def _init():
        acc_ref[pl.ds(b, 1), :] = part

    @pl.when(r != 0)
    def _acc():
        acc_ref[pl.ds(b, 1), :] = acc_ref[pl.ds(b, 1), :] + part

    @pl.when((b == _B - 1) & (r == nr - 1))
    def _topk():
        wm = acc_ref[...].reshape(_B, 16, 128)
        flat = (jax.lax.broadcasted_iota(jnp.int32, (_B, 16, 128), 1) * 128
                + jax.lax.broadcasted_iota(jnp.int32, (_B, 16, 128), 2))
        kiota = jax.lax.broadcasted_iota(jnp.int32, (_B, _K), 1)
        idx_acc = jnp.zeros((_B, _K), jnp.int32)
        for k in range(_K):
            m = jnp.max(wm, axis=(1, 2), keepdims=True)
            j = jnp.min(jnp.where(wm == m, flat, _S), axis=(1, 2),
                        keepdims=True)
            idx_acc = jnp.where(kiota == k, j[:, :, 0], idx_acc)
            wm = jnp.where(flat == j, -jnp.inf, wm)
        idx_ref[:, 0, :] = idx_acc


_sc_mesh = plsc.VectorSubcoreMesh(core_axis_name="c", subcore_axis_name="s")


@functools.partial(
    pl.kernel,
    mesh=_sc_mesh,
    out_type=jax.ShapeDtypeStruct((_B * _F * _K,), jnp.float32),
    scratch_types=(
        pltpu.VMEM((_K,), jnp.int32),
        pltpu.VMEM((_CHUNK, _S), jnp.float32),
        pltpu.VMEM((_CHUNK, _S), jnp.float32),
        pltpu.VMEM((_ROWS_PER_TILE * _K,), jnp.float32),
        pltpu.SemaphoreType.DMA,
        pltpu.SemaphoreType.DMA,
    ),
    compiler_params=pltpu.CompilerParams(use_tc_tiling_on_sc=False, needs_layout_passes=False),
)
def _sc_gather(x_hbm, idx_hbm, out_hbm, idxv, buf0, buf1, outbuf, sem0, sem1):
    wid = lax.axis_index("s") * 2 + lax.axis_index("c")
    r0 = wid * _ROWS_PER_TILE
    b = r0 // _F
    pltpu.sync_copy(idx_hbm.at[b], idxv)

    bufs = (buf0, buf1)
    sems = (sem0, sem1)
    nchunks = _ROWS_PER_TILE // _CHUNK

    def start(c):
        return pltpu.async_copy(
            x_hbm.at[pl.ds(r0 + c * _CHUNK, _CHUNK), :],
            bufs[c % 2], sems[c % 2])

    cp = start(0)
    for c in range(nchunks):
        nxt = start(c + 1) if c + 1 < nchunks else None
        cp.wait()
        buf = bufs[c % 2]
        for i in range(_CHUNK):
            rows = jnp.full((16,), i, jnp.int32)
            obase = (c * _CHUNK + i) * _K
            for q in range(_K // 16):
                cols = idxv[pl.ds(q * 16, 16)]
                outbuf[pl.ds(obase + q * 16, 16)] = plsc.load_gather(
                    buf, [rows, cols])
        cp = nxt
    pltpu.sync_copy(outbuf, out_hbm.at[pl.ds(r0 * _K, _ROWS_PER_TILE * _K)])


def kernel(x, w):
    idx3 = pl.pallas_call(
        _mean_topk_body,
        grid=(_B, _S // _BR),
        in_specs=[pl.BlockSpec((1, _BR, _S), lambda b, r: (b, r, 0))],
        out_specs=pl.BlockSpec((_B, 1, _K), lambda b, r: (0, 0, 0)),
        out_shape=jax.ShapeDtypeStruct((_B, 1, _K), jnp.int32),
        scratch_shapes=[pltpu.VMEM((_B, _S), jnp.float32)],
    )(w)

    idx = idx3.reshape(_B, _K)
    x2d = x.reshape(_B * _F, _S)
    out = _sc_gather(x2d, idx)
    return out.reshape(_B, _F, _K)


# speedup vs baseline: 1.8927x; 1.8927x over previous
import jax
import jax.numpy as jnp
from jax.experimental import pallas as pl
from jax.experimental.pallas import tpu as pltpu

_B, _S, _F, _K = 4, 2048, 1024, 64
_BR = 512
_BF = 1024


def _cumsum_lanes(x):
    n = x.shape[-1]
    d = 1
    while d < n:
        shifted = jnp.concatenate(
            [jnp.zeros_like(x[..., :d]), x[..., :-d]], axis=-1)
        x = x + shifted
        d *= 2
    return x


def _topk_idx(wm):
    s = jax.lax.bitcast_convert_type(wm, jnp.int32)
    k32 = jnp.where(s < 0, jnp.int32(-2147483648) - s, s)
    ku = k32 ^ jnp.int32(-2147483648)
    lane = jax.lax.broadcasted_iota(jnp.int32, (_B, _S), 1)

    active = jnp.ones((_B, _S), jnp.bool_)
    maskgt = jnp.zeros((_B, _S), jnp.bool_)
    g = jnp.zeros((_B, 1), jnp.int32)
    diota = jax.lax.broadcasted_iota(jnp.int32, (_B, 128, 1), 1)
    d128 = jax.lax.broadcasted_iota(jnp.int32, (_B, 128), 1)
    for shift in (28, 21, 14, 7, 0):
        dig = jax.lax.shift_right_logical(ku, shift) & 127
        oneh = ((dig[:, None, :] == diota)
                & active[:, None, :]).astype(jnp.int32)
        cnt = jnp.sum(oneh, axis=2)
        cgt = (jnp.sum(cnt, axis=1, keepdims=True)
               - _cumsum_lanes(cnt))
        gt_if = g + cgt
        cond = (gt_if < _K) & (gt_if + cnt >= _K)
        d_ = jnp.sum(jnp.where(cond, d128, 0), axis=1, keepdims=True)
        g = jnp.sum(jnp.where(cond, gt_if, 0), axis=1, keepdims=True)
        maskgt = maskgt | (active & (dig > d_))
        active = active & (dig == d_)

    need = _K - g
    eqrank = _cumsum_lanes(active.astype(jnp.int32))
    sel = maskgt | (active & (eqrank <= need))
    cpos = _cumsum_lanes(sel.astype(jnp.int32)) - 1

    kio_s = jax.lax.broadcasted_iota(jnp.int32, (_B, _K, 1), 1)
    ohT = ((cpos[:, None, :] == kio_s) & sel[:, None, :])
    vc = jnp.sum(jnp.where(ohT, wm[:, None, :], 0.0), axis=2,
                 keepdims=True)
    ic = jnp.sum(jnp.where(ohT, lane[:, None, :], 0), axis=2,
                 keepdims=True)

    vcl = vc.reshape(_B, 1, _K)
    icl = ic.reshape(_B, 1, _K)
    gtm = vcl > vc
    eqm = (vcl == vc) & (icl < ic)
    r = jnp.sum((gtm | eqm).astype(jnp.int32), axis=2, keepdims=True)

    kio_l = jax.lax.broadcasted_iota(jnp.int32, (_B, 1, _K), 2)
    idx = jnp.sum(jnp.where(r == kio_l, ic, 0), axis=1)
    return idx


def _mean_topk_body(w_ref, idx_ref, acc_ref):
    b = pl.program_id(0)
    r = pl.program_id(1)
    nr = pl.num_programs(1)
    part = jnp.sum(w_ref[0], axis=0, keepdims=True)

    @pl.when(r == 0)
    def _init():
        acc_ref[pl.ds(b, 1), :] = part

    @pl.when(r != 0)
    def _acc():
        acc_ref[pl.ds(b, 1), :] = acc_ref[pl.ds(b, 1), :] + part

    @pl.when((b == _B - 1) & (r == nr - 1))
    def _topk():
        idx_ref[:, 0, :] = _topk_idx(acc_ref[...])


def _gather_body(idx_ref, x_ref, out_ref):
    idx = idx_ref[0]
    onehot = (jax.lax.broadcasted_iota(jnp.int32, (_S, _K), 0) == idx
              ).astype(jnp.float32)
    out_ref[0] = jnp.dot(x_ref[0], onehot,
                         preferred_element_type=jnp.float32)


def kernel(x, w):
    idx3 = pl.pallas_call(
        _mean_topk_body,
        grid=(_B, _S // _BR),
        in_specs=[pl.BlockSpec((1, _BR, _S), lambda b, r: (b, r, 0))],
        out_specs=pl.BlockSpec((_B, 1, _K), lambda b, r: (0, 0, 0)),
        out_shape=jax.ShapeDtypeStruct((_B, 1, _K), jnp.int32),
        scratch_shapes=[pltpu.VMEM((_B, _S), jnp.float32)],
    )(w)

    out = pl.pallas_call(
        _gather_body,
        grid=(_B, _F // _BF),
        in_specs=[
            pl.BlockSpec((1, 1, _K), lambda b, f: (b, 0, 0)),
            pl.BlockSpec((1, _BF, _S), lambda b, f: (b, f, 0)),
        ],
        out_specs=pl.BlockSpec((1, _BF, _K), lambda b, f: (b, f, 0)),
        out_shape=jax.ShapeDtypeStruct((_B, _F, _K), jnp.float32),
    )(idx3, x)
    return out


# speedup vs baseline: 2.2103x; 1.1678x over previous
import jax
import jax.numpy as jnp
from jax.experimental import pallas as pl
from jax.experimental.pallas import tpu as pltpu

_B, _S, _F, _K = 4, 2048, 1024, 64
_BR = 512
_BF = 1024


def _cumsum_lanes(x):
    n = x.shape[-1]
    d = 1
    while d < n:
        shifted = jnp.concatenate(
            [jnp.zeros_like(x[..., :d]), x[..., :-d]], axis=-1)
        x = x + shifted
        d *= 2
    return x


def _topk_idx(wm):
    s = jax.lax.bitcast_convert_type(wm, jnp.int32)
    k32 = jnp.where(s < 0, jnp.int32(-2147483648) - s, s)
    lane = jax.lax.broadcasted_iota(jnp.int32, (_B, _S), 1)

    prefix = jnp.full((_B, 1), jnp.int32(-2147483648))
    for bit in range(31, -1, -1):
        cand = prefix + jnp.int32(1 << bit if bit < 31 else -2147483648)
        cnt = jnp.sum((k32 >= cand).astype(jnp.int32), axis=1,
                      keepdims=True)
        prefix = jnp.where(cnt >= _K, cand, prefix)
    t = prefix
    maskgt = k32 > t
    active = k32 == t
    g = jnp.sum(maskgt.astype(jnp.int32), axis=1, keepdims=True)

    need = _K - g
    eqrank = _cumsum_lanes(active.astype(jnp.int32))
    sel = maskgt | (active & (eqrank <= need))
    cpos = _cumsum_lanes(sel.astype(jnp.int32)) - 1

    kio_s = jax.lax.broadcasted_iota(jnp.int32, (_B, _K, 1), 1)
    ohT = ((cpos[:, None, :] == kio_s) & sel[:, None, :])
    vc = jnp.sum(jnp.where(ohT, wm[:, None, :], 0.0), axis=2,
                 keepdims=True)
    ic = jnp.sum(jnp.where(ohT, lane[:, None, :], 0), axis=2,
                 keepdims=True)

    vcl = vc.reshape(_B, 1, _K)
    icl = ic.reshape(_B, 1, _K)
    gtm = vcl > vc
    eqm = (vcl == vc) & (icl < ic)
    r = jnp.sum((gtm | eqm).astype(jnp.int32), axis=2, keepdims=True)

    kio_l = jax.lax.broadcasted_iota(jnp.int32, (_B, 1, _K), 2)
    idx = jnp.sum(jnp.where(r == kio_l, ic, 0), axis=1)
    return idx


def _mean_topk_body(w_ref, idx_ref, acc_ref):
    b = pl.program_id(0)
    r = pl.program_id(1)
    nr = pl.num_programs(1)
    part = jnp.sum(w_ref[0], axis=0, keepdims=True)

    @pl.when(r == 0)
    def _init():
        acc_ref[pl.ds(b, 1), :] = part

    @pl.when(r != 0)
    def _acc():
        acc_ref[pl.ds(b, 1), :] = acc_ref[pl.ds(b, 1), :] + part

    @pl.when((b == _B - 1) & (r == nr - 1))
    def _topk():
        idx_ref[:, 0, :] = _topk_idx(acc_ref[...])


def _gather_body(idx_ref, x_ref, out_ref):
    idx = idx_ref[0]
    onehot = (jax.lax.broadcasted_iota(jnp.int32, (_S, _K), 0) == idx
              ).astype(jnp.float32)
    out_ref[0] = jnp.dot(x_ref[0], onehot,
                         preferred_element_type=jnp.float32)


def kernel(x, w):
    idx3 = pl.pallas_call(
        _mean_topk_body,
        grid=(_B, _S // _BR),
        in_specs=[pl.BlockSpec((1, _BR, _S), lambda b, r: (b, r, 0))],
        out_specs=pl.BlockSpec((_B, 1, _K), lambda b, r: (0, 0, 0)),
        out_shape=jax.ShapeDtypeStruct((_B, 1, _K), jnp.int32),
        scratch_shapes=[pltpu.VMEM((_B, _S), jnp.float32)],
    )(w)

    out = pl.pallas_call(
        _gather_body,
        grid=(_B, _F // _BF),
        in_specs=[
            pl.BlockSpec((1, 1, _K), lambda b, f: (b, 0, 0)),
            pl.BlockSpec((1, _BF, _S), lambda b, f: (b, f, 0)),
        ],
        out_specs=pl.BlockSpec((1, _BF, _K), lambda b, f: (b, f, 0)),
        out_shape=jax.ShapeDtypeStruct((_B, _F, _K), jnp.float32),
    )(idx3, x)
    return out
